# baseline (speedup 1.0000x reference)
import jax
import jax.numpy as jnp
from jax import lax
from jax.experimental import pallas as pl
from jax.experimental.pallas import tpu as pltpu
from jax.experimental.pallas import tpu_sc as plsc

MEM = 10000
B = 2048
NCL = 100
ISZ = (3, 32, 32)
PK = 128
NC = 2
NS = 16
NT = NC * NS
NB = B // 16
CA = 8


def _lane(vec, k):
    return vec[k]


def _body(xw, xs, xp, sw, ss, bp_in, rand_hbm,
          bw, bpk, rv, pm, pb, idxb, idxm, rsem, wsem):
    cax = lax.axis_index("c")
    sax = lax.axis_index("s")
    w = sax * NC + cax

    pltpu.sync_copy(rand_hbm, rv)
    li = lax.iota(jnp.int32, 16)

    def scan(ci, cnt):
        base = pl.multiple_of(ci * 16, 16)
        r = rv[pl.ds(base, 16)]
        hit = (r < MEM) & (((r >> 4) & (NT - 1)) == w)
        nh = _lane(plsc.all_reduce_population_count(hit), 0)

        @pl.when(nh > 0)
        def _append():
            h32 = jnp.where(hit, 1, 0)
            rank = li * 0
            for k in range(15):
                rank = rank + jnp.where((li > k) & (_lane(h32, k) > 0), 1, 0)
            plsc.store_scatter(pm.at[...], [cnt + rank], r, mask=hit)
            plsc.store_scatter(pb.at[...], [cnt + rank], li + ci * 16,
                               mask=hit)

        return cnt + nh

    cnt = lax.fori_loop(0, NB, scan, jnp.int32(0))
    nchunks = (cnt + CA - 1) // CA

    lo8 = li < CA

    def apply(t, carry):
        o = pl.multiple_of(t * CA, CA)
        mv = pm[pl.ds(o, 16)]
        bv = pb[pl.ds(o, 16)]
        vc = jnp.minimum(cnt - o, CA)
        mvp = jnp.where(li < vc, mv, _lane(mv, 0))
        bvp = jnp.where(li < vc, bv, _lane(bv, 0))
        dup = li < 0
        for k in range(CA - 1):
            dup = dup | ((mvp == (li * 0 + _lane(mv, k))) & (li > k)
                         & (li < vc) & (k < vc))
        ndup = _lane(plsc.all_reduce_population_count(dup), 0)

        plsc.store_scatter(idxm.at[...], [li], mvp, mask=lo8)
        plsc.store_scatter(idxb.at[...], [li], bvp, mask=lo8)

        @pl.when(ndup == 0)
        def _fast():
            pltpu.async_copy(bp_in.at[idxb], bpk, rsem).wait()
            pltpu.async_copy(bpk, xp.at[idxm], wsem).wait()
            for src, dst in ((sw, xw), (ss, xs)):
                gts = [pltpu.async_copy(src.at[pl.ds(_lane(bvp, k), 1)],
                                        bw.at[pl.ds(k, 1)], rsem)
                       for k in range(CA)]
                for cp in gts:
                    cp.wait()
                sts = [pltpu.async_copy(bw.at[pl.ds(k, 1)],
                                        dst.at[pl.ds(_lane(mvp, k), 1)], wsem)
                       for k in range(CA)]
                for cp in sts:
                    cp.wait()

        @pl.when(ndup > 0)
        def _fallback():
            for k in range(CA):
                mk = _lane(mvp, k)
                bk = _lane(bvp, k)
                plsc.store_scatter(idxm.at[...], [li], li * 0 + mk, mask=lo8)
                plsc.store_scatter(idxb.at[...], [li], li * 0 + bk, mask=lo8)
                pltpu.async_copy(bp_in.at[idxb], bpk, rsem).wait()
                pltpu.async_copy(bpk, xp.at[idxm], wsem).wait()
                for src, dst in ((sw, xw), (ss, xs)):
                    pltpu.async_copy(src.at[pl.ds(bk, 1)],
                                     bw.at[pl.ds(0, 1)], rsem).wait()
                    pltpu.async_copy(bw.at[pl.ds(0, 1)],
                                     dst.at[pl.ds(mk, 1)], wsem).wait()

        return carry

    lax.fori_loop(0, nchunks, apply, 0)


def kernel(mem_weak, mem_strong, mem_label, mem_partial, mem_task, mem_index,
           sample_weak, sample_strong, label, partial, task, index, rand_idx):
    i32 = jnp.int32
    f32 = jnp.float32

    mp_pad = jnp.concatenate(
        [mem_partial, mem_label[:, None], mem_task[:, None],
         mem_index[:, None], jnp.zeros((MEM, PK - NCL - 3), i32)], axis=1)
    taskcol = jnp.full((B,), task, i32)
    bp_pad = jnp.concatenate(
        [partial, label[:, None], taskcol[:, None],
         index[:, None], jnp.zeros((B, PK - NCL - 3), i32)], axis=1)

    xw = jax.new_ref(mem_weak)
    xs = jax.new_ref(mem_strong)
    xp = jax.new_ref(mp_pad)

    scatter = pl.kernel(
        _body,
        out_type=(),
        mesh=plsc.VectorSubcoreMesh(core_axis_name="c", subcore_axis_name="s"),
        compiler_params=pltpu.CompilerParams(needs_layout_passes=False),
        scratch_types=[
            pltpu.VMEM((CA,) + ISZ, f32),
            pltpu.VMEM((CA, PK), i32),
            pltpu.VMEM((B,), i32),
            pltpu.VMEM((B + 16,), i32),
            pltpu.VMEM((B + 16,), i32),
            pltpu.VMEM((CA,), i32),
            pltpu.VMEM((CA,), i32),
            pltpu.SemaphoreType.DMA,
            pltpu.SemaphoreType.DMA,
        ],
    )
    scatter(xw, xs, xp, sample_weak, sample_strong, bp_pad, rand_idx)

    ow = xw[...]
    os_ = xs[...]
    op_ = xp[...]
    return (ow, os_,
            op_[:, NCL], op_[:, :NCL], op_[:, NCL + 1], op_[:, NCL + 2])


# speedup vs baseline: 2.8869x; 2.8869x over previous
import jax
import jax.numpy as jnp
from jax import lax
from jax.experimental import pallas as pl
from jax.experimental.pallas import tpu as pltpu
from jax.experimental.pallas import tpu_sc as plsc

MEM = 10000
B = 2048
NCL = 100
D = 3 * 32 * 32
PK = 128
NC = 2
NS = 16
NT = NC * NS
NB = B // 16
CA = 8


def _lane(vec, k):
    return vec[k]


def _body(xw, xs, xp, sw, ss, bp_in, rand_hbm,
          bw, bs, bpk, rv, pm, pb, idxb, idxm, rsem, wsem):
    cax = lax.axis_index("c")
    sax = lax.axis_index("s")
    w = sax * NC + cax

    pltpu.sync_copy(rand_hbm, rv)
    li = lax.iota(jnp.int32, 16)

    def scan(ci, cnt):
        base = pl.multiple_of(ci * 16, 16)
        r = rv[pl.ds(base, 16)]
        hit = (r < MEM) & (((r >> 4) & (NT - 1)) == w)
        nh = _lane(plsc.all_reduce_population_count(hit), 0)

        @pl.when(nh > 0)
        def _append():
            h32 = jnp.where(hit, 1, 0)
            rank = li * 0
            for k in range(15):
                rank = rank + jnp.where((li > k) & (_lane(h32, k) > 0), 1, 0)
            plsc.store_scatter(pm.at[...], [cnt + rank], r, mask=hit)
            plsc.store_scatter(pb.at[...], [cnt + rank], li + ci * 16,
                               mask=hit)

        return cnt + nh

    cnt = lax.fori_loop(0, NB, scan, jnp.int32(0))
    nchunks = (cnt + CA - 1) // CA

    lo8 = li < CA

    def apply_chunk():
        gts = (
            pltpu.async_copy(sw.at[idxb], bw, rsem),
            pltpu.async_copy(ss.at[idxb], bs, rsem),
            pltpu.async_copy(bp_in.at[idxb], bpk, rsem),
        )
        for cp in gts:
            cp.wait()
        sts = (
            pltpu.async_copy(bw, xw.at[idxm], wsem),
            pltpu.async_copy(bs, xs.at[idxm], wsem),
            pltpu.async_copy(bpk, xp.at[idxm], wsem),
        )
        for cp in sts:
            cp.wait()

    def apply(t, carry):
        o = pl.multiple_of(t * CA, CA)
        mv = pm[pl.ds(o, 16)]
        bv = pb[pl.ds(o, 16)]
        vc = jnp.minimum(cnt - o, CA)
        mvp = jnp.where(li < vc, mv, _lane(mv, 0))
        bvp = jnp.where(li < vc, bv, _lane(bv, 0))
        dup = li < 0
        for k in range(CA - 1):
            dup = dup | ((mvp == (li * 0 + _lane(mv, k))) & (li > k)
                         & (li < vc) & (k < vc))
        ndup = _lane(plsc.all_reduce_population_count(dup), 0)

        @pl.when(ndup == 0)
        def _fast():
            plsc.store_scatter(idxm.at[...], [li], mvp, mask=lo8)
            plsc.store_scatter(idxb.at[...], [li], bvp, mask=lo8)
            apply_chunk()

        @pl.when(ndup > 0)
        def _fallback():
            for k in range(CA):
                @pl.when(k < vc)
                def _one(k=k):
                    plsc.store_scatter(idxm.at[...], [li],
                                       li * 0 + _lane(mvp, k), mask=lo8)
                    plsc.store_scatter(idxb.at[...], [li],
                                       li * 0 + _lane(bvp, k), mask=lo8)
                    apply_chunk()

        return carry

    lax.fori_loop(0, nchunks, apply, 0)


def kernel(mem_weak, mem_strong, mem_label, mem_partial, mem_task, mem_index,
           sample_weak, sample_strong, label, partial, task, index, rand_idx):
    i32 = jnp.int32
    f32 = jnp.float32
    sw2 = sample_weak.reshape(B, D)
    ss2 = sample_strong.reshape(B, D)

    mp_pad = jnp.concatenate(
        [mem_partial, mem_label[:, None], mem_task[:, None],
         mem_index[:, None], jnp.zeros((MEM, PK - NCL - 3), i32)], axis=1)
    taskcol = jnp.full((B,), task, i32)
    bp_pad = jnp.concatenate(
        [partial, label[:, None], taskcol[:, None],
         index[:, None], jnp.zeros((B, PK - NCL - 3), i32)], axis=1)

    xw = jax.new_ref(mem_weak.reshape(MEM, D))
    xs = jax.new_ref(mem_strong.reshape(MEM, D))
    xp = jax.new_ref(mp_pad)

    scatter = pl.kernel(
        _body,
        out_type=(),
        mesh=plsc.VectorSubcoreMesh(core_axis_name="c", subcore_axis_name="s"),
        compiler_params=pltpu.CompilerParams(needs_layout_passes=False),
        scratch_types=[
            pltpu.VMEM((CA, D), f32),
            pltpu.VMEM((CA, D), f32),
            pltpu.VMEM((CA, PK), i32),
            pltpu.VMEM((B,), i32),
            pltpu.VMEM((B + 16,), i32),
            pltpu.VMEM((B + 16,), i32),
            pltpu.VMEM((CA,), i32),
            pltpu.VMEM((CA,), i32),
            pltpu.SemaphoreType.DMA,
            pltpu.SemaphoreType.DMA,
        ],
    )
    scatter(xw, xs, xp, sw2, ss2, bp_pad, rand_idx)

    ow = xw[...]
    os_ = xs[...]
    op_ = xp[...]
    return (ow.reshape(mem_weak.shape), os_.reshape(mem_strong.shape),
            op_[:, NCL], op_[:, :NCL], op_[:, NCL + 1], op_[:, NCL + 2])


# speedup vs baseline: 3.0568x; 1.0588x over previous
import jax
import jax.numpy as jnp
from jax import lax
from jax.experimental import pallas as pl
from jax.experimental.pallas import tpu as pltpu
from jax.experimental.pallas import tpu_sc as plsc

MEM = 10000
B = 2048
NCL = 100
D = 3 * 32 * 32
PK = 128
NC = 2
NS = 16
NT = NC * NS
NB = B // 16
CA = 8


def _lane(vec, k):
    return vec[k]


def _make_body(n_big, has_packed):
    def body(*args):
        a = list(args)
        xbig = [a.pop(0) for _ in range(n_big)]
        xp = a.pop(0) if has_packed else None
        sbig = [a.pop(0) for _ in range(n_big)]
        bp_in = a.pop(0) if has_packed else None
        rand_hbm = a.pop(0)
        bbig = [a.pop(0) for _ in range(n_big)]
        bpk = a.pop(0) if has_packed else None
        rv, pm, pb, idxb, idxm, rsem, wsem = a

        cax = lax.axis_index("c")
        sax = lax.axis_index("s")
        w = sax * NC + cax

        pltpu.sync_copy(rand_hbm, rv)
        li = lax.iota(jnp.int32, 16)

        def scan(ci, cnt):
            base = pl.multiple_of(ci * 16, 16)
            r = rv[pl.ds(base, 16)]
            hit = (r < MEM) & (((r >> 4) & (NT - 1)) == w)
            nh = _lane(plsc.all_reduce_population_count(hit), 0)

            @pl.when(nh > 0)
            def _append():
                h32 = jnp.where(hit, 1, 0)
                rank = li * 0
                for k in range(15):
                    rank = rank + jnp.where((li > k) & (_lane(h32, k) > 0),
                                            1, 0)
                plsc.store_scatter(pm.at[...], [cnt + rank], r, mask=hit)
                plsc.store_scatter(pb.at[...], [cnt + rank], li + ci * 16,
                                   mask=hit)

            return cnt + nh

        cnt = lax.fori_loop(0, NB, scan, jnp.int32(0))
        nchunks = (cnt + CA - 1) // CA
        lo8 = li < CA

        def apply_chunk():
            gts = [pltpu.async_copy(s.at[idxb], b, rsem)
                   for s, b in zip(sbig, bbig)]
            if has_packed:
                gts.append(pltpu.async_copy(bp_in.at[idxb], bpk, rsem))
            for cp in gts:
                cp.wait()
            sts = [pltpu.async_copy(b, x.at[idxm], wsem)
                   for x, b in zip(xbig, bbig)]
            if has_packed:
                sts.append(pltpu.async_copy(bpk, xp.at[idxm], wsem))
            for cp in sts:
                cp.wait()

        def apply(t, carry):
            o = pl.multiple_of(t * CA, CA)
            mv = pm[pl.ds(o, 16)]
            bv = pb[pl.ds(o, 16)]
            vc = jnp.minimum(cnt - o, CA)
            mvp = jnp.where(li < vc, mv, _lane(mv, 0))
            bvp = jnp.where(li < vc, bv, _lane(bv, 0))
            dup = li < 0
            for k in range(CA - 1):
                dup = dup | ((mvp == (li * 0 + _lane(mv, k))) & (li > k)
                             & (li < vc) & (k < vc))
            ndup = _lane(plsc.all_reduce_population_count(dup), 0)

            @pl.when(ndup == 0)
            def _fast():
                plsc.store_scatter(idxm.at[...], [li], mvp, mask=lo8)
                plsc.store_scatter(idxb.at[...], [li], bvp, mask=lo8)
                apply_chunk()

            @pl.when(ndup > 0)
            def _fallback():
                for k in range(CA):
                    @pl.when(k < vc)
                    def _one(k=k):
                        plsc.store_scatter(idxm.at[...], [li],
                                           li * 0 + _lane(mvp, k), mask=lo8)
                        plsc.store_scatter(idxb.at[...], [li],
                                           li * 0 + _lane(bvp, k), mask=lo8)
                        apply_chunk()

            return carry

        lax.fori_loop(0, nchunks, apply, 0)

    return body


def _scatter_call(n_big, has_packed):
    f32 = jnp.float32
    i32 = jnp.int32
    scratch = [pltpu.VMEM((CA, D), f32) for _ in range(n_big)]
    if has_packed:
        scratch.append(pltpu.VMEM((CA, PK), i32))
    scratch += [
        pltpu.VMEM((B,), i32),
        pltpu.VMEM((B + 16,), i32),
        pltpu.VMEM((B + 16,), i32),
        pltpu.VMEM((CA,), i32),
        pltpu.VMEM((CA,), i32),
        pltpu.SemaphoreType.DMA,
        pltpu.SemaphoreType.DMA,
    ]
    return pl.kernel(
        _make_body(n_big, has_packed),
        out_type=(),
        mesh=plsc.VectorSubcoreMesh(core_axis_name="c", subcore_axis_name="s"),
        compiler_params=pltpu.CompilerParams(needs_layout_passes=False),
        scratch_types=scratch,
    )


def kernel(mem_weak, mem_strong, mem_label, mem_partial, mem_task, mem_index,
           sample_weak, sample_strong, label, partial, task, index, rand_idx):
    i32 = jnp.int32
    f32 = jnp.float32
    sw2 = sample_weak.reshape(B, D)
    ss2 = sample_strong.reshape(B, D)

    mp_pad = jnp.concatenate(
        [mem_partial, mem_label[:, None], mem_task[:, None],
         mem_index[:, None], jnp.zeros((MEM, PK - NCL - 3), i32)], axis=1)
    taskcol = jnp.full((B,), task, i32)
    bp_pad = jnp.concatenate(
        [partial, label[:, None], taskcol[:, None],
         index[:, None], jnp.zeros((B, PK - NCL - 3), i32)], axis=1)

    xw = jax.new_ref(mem_weak.reshape(MEM, D))
    xs = jax.new_ref(mem_strong.reshape(MEM, D))
    xp = jax.new_ref(mp_pad)

    _scatter_call(1, False)(xw, sw2, rand_idx)
    _scatter_call(1, True)(xs, xp, ss2, bp_pad, rand_idx)

    ow = xw[...]
    os_ = xs[...]
    op_ = xp[...]
    return (ow.reshape(mem_weak.shape), os_.reshape(mem_strong.shape),
            op_[:, NCL], op_[:, :NCL], op_[:, NCL + 1], op_[:, NCL + 2])
